# baseline (speedup 1.0000x reference)
import functools

import jax
import jax.numpy as jnp
from jax import lax
from jax.experimental import pallas as pl
from jax.experimental.pallas import tpu as pltpu, tpu_sc as plsc

D_HALF = 64
D = 128
NV = D_HALF * D_HALF
B, S = 4096, 50
N_TOTAL = B * S
NC, NS, L = 2, 16, 16
NW = NC * NS
B_PER_W = B // NW
N_PER_W = N_TOTAL // NW
GROUPS = N_PER_W // L
NBUF = 8
N_WAVES = B_PER_W // NBUF


def _idx_body(coords_hbm, idx_hbm, coords_v, idx_v):
    wid = lax.axis_index("s") * NC + lax.axis_index("c")
    base_b = wid * B_PER_W

    pltpu.sync_copy(coords_hbm.at[pl.ds(base_b, B_PER_W)], coords_v)

    iota = lax.iota(jnp.int32, L)
    zero = iota * 0
    one = zero + 1

    def prep_body(g, carry):
        nvec = g * L + iota
        q = nvec // S
        m = nvec % S
        r = plsc.load_gather(coords_v, [q, m, zero])
        c = plsc.load_gather(coords_v, [q, m, one])
        v = (jnp.clip(r, 0, D_HALF - 1) * D_HALF
             + jnp.clip(c, 0, D_HALF - 1))
        plsc.store_scatter(idx_v, [q, m], v)
        return carry

    lax.fori_loop(0, GROUPS, prep_body, 0)
    pltpu.sync_copy(idx_v, idx_hbm.at[pl.ds(base_b, B_PER_W)])


_make_idx = functools.partial(
    pl.kernel,
    out_type=jax.ShapeDtypeStruct((B, D), jnp.int32),
    mesh=plsc.VectorSubcoreMesh(core_axis_name="c", subcore_axis_name="s"),
    scratch_types=[
        pltpu.VMEM((B_PER_W, S, 2), jnp.int32),
        pltpu.VMEM((B_PER_W, D), jnp.int32),
    ],
    compiler_params=pltpu.CompilerParams(
        needs_layout_passes=False, use_tc_tiling_on_sc=False),
)(_idx_body)


def _comb_body(rt_ref, ct_ref, out_ref):
    i0 = lax.broadcasted_iota(jnp.int32, (NV, D_HALF), 0)
    i1 = lax.broadcasted_iota(jnp.int32, (NV, D_HALF), 1)
    oh_r = (i0 // D_HALF == i1).astype(jnp.float32)
    oh_c = (i0 % D_HALF == i1).astype(jnp.float32)
    out_ref[:, :D_HALF] = jnp.dot(oh_r, rt_ref[...],
                                  preferred_element_type=jnp.float32)
    out_ref[:, D_HALF:] = jnp.dot(oh_c, ct_ref[...],
                                  preferred_element_type=jnp.float32)


_build_comb = pl.pallas_call(
    _comb_body,
    out_shape=jax.ShapeDtypeStruct((NV, D), jnp.float32),
)


def _body(idx_hbm, comb_hbm, out_hbm, idx_v, *rest):
    bufs = rest[:NBUF]
    gsem = rest[NBUF:2 * NBUF]
    osem = rest[2 * NBUF:3 * NBUF]

    wid = lax.axis_index("s") * NC + lax.axis_index("c")
    base_b = wid * B_PER_W

    pltpu.sync_copy(idx_hbm.at[pl.ds(base_b, B_PER_W)], idx_v)

    def wave_body(i, carry):
        handles = []
        for b in range(NBUF):
            ci = i * NBUF + b

            @pl.when(i > 0)
            def _wait_out(b=b):
                pltpu.make_async_copy(
                    bufs[b], out_hbm.at[base_b], osem[b]).wait()

            handles.append(
                pltpu.async_copy(
                    comb_hbm.at[idx_v.at[ci, pl.ds(0, S)]], bufs[b], gsem[b]))
        for b in range(NBUF):
            ci = i * NBUF + b
            handles[b].wait()
            pltpu.async_copy(bufs[b], out_hbm.at[base_b + ci], osem[b])
        return carry

    lax.fori_loop(0, N_WAVES, wave_body, 0)

    for b in range(NBUF):
        pltpu.make_async_copy(bufs[b], out_hbm.at[base_b], osem[b]).wait()


_encode = functools.partial(
    pl.kernel,
    out_type=jax.ShapeDtypeStruct((B, S, D), jnp.float32),
    mesh=plsc.VectorSubcoreMesh(core_axis_name="c", subcore_axis_name="s"),
    scratch_types=(
        [pltpu.VMEM((B_PER_W, D), jnp.int32)]
        + [pltpu.VMEM((S, D), jnp.float32) for _ in range(NBUF)]
        + [pltpu.SemaphoreType.DMA for _ in range(2 * NBUF)]
    ),
    compiler_params=pltpu.CompilerParams(
        needs_layout_passes=False, use_tc_tiling_on_sc=True),
)(_body)


def kernel(coords, row_emb, col_emb):
    idx = _make_idx(coords.astype(jnp.int32))
    comb = _build_comb(row_emb, col_emb)
    return _encode(idx, comb)


# speedup vs baseline: 2.0311x; 2.0311x over previous
import functools

import jax
import jax.numpy as jnp
from jax import lax
from jax.experimental import pallas as pl
from jax.experimental.pallas import tpu as pltpu, tpu_sc as plsc

D_HALF = 64
D = 128
NV = D_HALF * D_HALF
B, S = 4096, 50
N_TOTAL = B * S
NC, NS, L = 2, 16, 16
NW = NC * NS
B_PER_W = B // NW
N_PER_W = N_TOTAL // NW
GROUPS = N_PER_W // L
NBUF = 8
N_WAVES = B_PER_W // NBUF


def _comb_body(rt_ref, ct_ref, out_ref):
    i0 = lax.broadcasted_iota(jnp.int32, (NV, D_HALF), 0)
    i1 = lax.broadcasted_iota(jnp.int32, (NV, D_HALF), 1)
    oh_r = (i0 // D_HALF == i1).astype(jnp.float32)
    oh_c = (i0 % D_HALF == i1).astype(jnp.float32)
    out_ref[:, :D_HALF] = jnp.dot(oh_r, rt_ref[...],
                                  preferred_element_type=jnp.float32)
    out_ref[:, D_HALF:] = jnp.dot(oh_c, ct_ref[...],
                                  preferred_element_type=jnp.float32)


_build_comb = pl.pallas_call(
    _comb_body,
    out_shape=jax.ShapeDtypeStruct((NV, D), jnp.float32),
)


def _body(idx_hbm, comb_hbm, out_hbm, idx_v, *rest):
    bufs = rest[:NBUF]
    gsem = rest[NBUF:2 * NBUF]
    osem = rest[2 * NBUF:3 * NBUF]

    wid = lax.axis_index("s") * NC + lax.axis_index("c")
    base_b = wid * B_PER_W

    pltpu.sync_copy(idx_hbm.at[pl.ds(base_b, B_PER_W)], idx_v)

    def wave_body(i, carry):
        handles = []
        for b in range(NBUF):
            ci = i * NBUF + b

            @pl.when(i > 0)
            def _wait_out(b=b):
                pltpu.make_async_copy(
                    bufs[b], out_hbm.at[base_b], osem[b]).wait()

            handles.append(
                pltpu.async_copy(
                    comb_hbm.at[idx_v.at[ci]], bufs[b], gsem[b]))
        for b in range(NBUF):
            ci = i * NBUF + b
            handles[b].wait()
            pltpu.async_copy(bufs[b], out_hbm.at[base_b + ci], osem[b])
        return carry

    lax.fori_loop(0, N_WAVES, wave_body, 0)

    for b in range(NBUF):
        pltpu.make_async_copy(bufs[b], out_hbm.at[base_b], osem[b]).wait()


_encode = functools.partial(
    pl.kernel,
    out_type=jax.ShapeDtypeStruct((B, S, D), jnp.float32),
    mesh=plsc.VectorSubcoreMesh(core_axis_name="c", subcore_axis_name="s"),
    scratch_types=(
        [pltpu.VMEM((B_PER_W, S), jnp.int32)]
        + [pltpu.VMEM((S, D), jnp.float32) for _ in range(NBUF)]
        + [pltpu.SemaphoreType.DMA for _ in range(2 * NBUF)]
    ),
    compiler_params=pltpu.CompilerParams(
        needs_layout_passes=False, use_tc_tiling_on_sc=True),
)(_body)


def kernel(coords, row_emb, col_emb):
    c32 = coords.astype(jnp.int32)
    idx = (jnp.clip(c32[..., 0], 0, D_HALF - 1) * D_HALF
           + jnp.clip(c32[..., 1], 0, D_HALF - 1))
    comb = _build_comb(row_emb, col_emb)
    return _encode(idx, comb)


# speedup vs baseline: 2.6030x; 1.2816x over previous
import functools

import jax
import jax.numpy as jnp
from jax import lax
from jax.experimental import pallas as pl
from jax.experimental.pallas import tpu as pltpu, tpu_sc as plsc

D_HALF = 64
D = 128
NV = D_HALF * D_HALF
B, S = 4096, 50
N_TOTAL = B * S
NC, NS, L = 2, 16, 16
NW = NC * NS
B_PER_W = B // NW
N_PER_W = N_TOTAL // NW
GROUPS = N_PER_W // L
NBUF = 8
N_WAVES = B_PER_W // NBUF


def _comb_body(rt_ref, ct_ref, out_ref):
    i0 = lax.broadcasted_iota(jnp.int32, (NV, D_HALF), 0)
    i1 = lax.broadcasted_iota(jnp.int32, (NV, D_HALF), 1)
    oh_r = (i0 // D_HALF == i1).astype(jnp.float32)
    oh_c = (i0 % D_HALF == i1).astype(jnp.float32)
    out_ref[:, :D_HALF] = jnp.dot(oh_r, rt_ref[...],
                                  preferred_element_type=jnp.float32)
    out_ref[:, D_HALF:] = jnp.dot(oh_c, ct_ref[...],
                                  preferred_element_type=jnp.float32)


_build_comb = pl.pallas_call(
    _comb_body,
    out_shape=jax.ShapeDtypeStruct((NV, D), jnp.float32),
)


def _body(idx_hbm, comb_hbm, out_hbm, idx_v, comb_sp, *rest):
    bufs = rest[:NBUF]
    gsem = rest[NBUF:2 * NBUF]
    osem = rest[2 * NBUF:3 * NBUF]

    sid = lax.axis_index("s")
    wid = sid * NC + lax.axis_index("c")
    base_b = wid * B_PER_W

    rows = NV // NS
    pltpu.sync_copy(comb_hbm.at[pl.ds(sid * rows, rows)],
                    comb_sp.at[pl.ds(sid * rows, rows)])
    pltpu.sync_copy(idx_hbm.at[pl.ds(base_b, B_PER_W)], idx_v)
    plsc.subcore_barrier()

    def wave_body(i, carry):
        handles = []
        for b in range(NBUF):
            ci = i * NBUF + b

            @pl.when(i > 0)
            def _wait_out(b=b):
                pltpu.make_async_copy(
                    bufs[b], out_hbm.at[base_b], osem[b]).wait()

            handles.append(
                pltpu.async_copy(
                    comb_sp.at[idx_v.at[ci]], bufs[b], gsem[b]))
        for b in range(NBUF):
            ci = i * NBUF + b
            handles[b].wait()
            pltpu.async_copy(bufs[b], out_hbm.at[base_b + ci], osem[b])
        return carry

    lax.fori_loop(0, N_WAVES, wave_body, 0)

    for b in range(NBUF):
        pltpu.make_async_copy(bufs[b], out_hbm.at[base_b], osem[b]).wait()


_encode = functools.partial(
    pl.kernel,
    out_type=jax.ShapeDtypeStruct((B, S, D), jnp.float32),
    mesh=plsc.VectorSubcoreMesh(core_axis_name="c", subcore_axis_name="s"),
    scratch_types=(
        [pltpu.VMEM((B_PER_W, S), jnp.int32),
         pltpu.VMEM_SHARED((NV, D), jnp.float32)]
        + [pltpu.VMEM((S, D), jnp.float32) for _ in range(NBUF)]
        + [pltpu.SemaphoreType.DMA for _ in range(2 * NBUF)]
    ),
    compiler_params=pltpu.CompilerParams(
        needs_layout_passes=False, use_tc_tiling_on_sc=True),
)(_body)


def kernel(coords, row_emb, col_emb):
    c32 = coords.astype(jnp.int32)
    idx = (jnp.clip(c32[..., 0], 0, D_HALF - 1) * D_HALF
           + jnp.clip(c32[..., 1], 0, D_HALF - 1))
    comb = _build_comb(row_emb, col_emb)
    return _encode(idx, comb)
